# Initial kernel scaffold; baseline (speedup 1.0000x reference)
import jax
import jax.numpy as jnp
from jax.experimental import pallas as pl


def kernel(verts, verts_features, skips, edge_index, params):
    raise NotImplementedError("write your pallas kernel here")


# speedup vs baseline: 1.0756x; 1.0756x over previous
import functools

import jax
import jax.numpy as jnp
from jax.experimental import pallas as pl
from jax.experimental.pallas import tpu as pltpu

V = 10000
E = 320000
HID = 128
N_EULER = 2
GRID = 32
SKIP_C = 32

_RB = 1000


def _seg_mean_sum(t, src, dst):
    msg = jnp.take(t, src, axis=0)
    return jax.ops.segment_sum(msg, dst, num_segments=V)


def _combine_kernel(x_ref, agg_ref, invd_ref, ws_ref, wn_ref, o_ref):
    agg = agg_ref[...] * invd_ref[...]
    o_ref[...] = jnp.maximum(
        jnp.dot(x_ref[...], ws_ref[...], preferred_element_type=jnp.float32)
        + jnp.dot(agg, wn_ref[...], preferred_element_type=jnp.float32), 0.0)


def _combine_res_kernel(x_ref, agg_ref, invd_ref, ws_ref, wn_ref, skip_ref,
                        o_ref):
    agg = agg_ref[...] * invd_ref[...]
    h = jnp.maximum(
        jnp.dot(x_ref[...], ws_ref[...], preferred_element_type=jnp.float32)
        + jnp.dot(agg, wn_ref[...], preferred_element_type=jnp.float32), 0.0)
    o_ref[...] = jnp.maximum(h + skip_ref[...], 0.0)


def _row_spec(c):
    return pl.BlockSpec((_RB, c), lambda i: (i, 0))


def _w_spec(r, c):
    return pl.BlockSpec((r, c), lambda i: (0, 0))


def _combine(x, agg, invd, ws, wn):
    cin = x.shape[1]
    ca = agg.shape[1]
    return pl.pallas_call(
        _combine_kernel,
        grid=(V // _RB,),
        in_specs=[_row_spec(cin), _row_spec(ca), _row_spec(1),
                  _w_spec(cin, HID), _w_spec(ca, HID)],
        out_specs=_row_spec(HID),
        out_shape=jax.ShapeDtypeStruct((V, HID), jnp.float32),
    )(x, agg, invd, ws, wn)


def _combine_res(x, agg, invd, ws, wn, skip):
    cin = x.shape[1]
    ca = agg.shape[1]
    return pl.pallas_call(
        _combine_res_kernel,
        grid=(V // _RB,),
        in_specs=[_row_spec(cin), _row_spec(ca), _row_spec(1),
                  _w_spec(cin, HID), _w_spec(ca, HID), _row_spec(HID)],
        out_specs=_row_spec(HID),
        out_shape=jax.ShapeDtypeStruct((V, HID), jnp.float32),
    )(x, agg, invd, ws, wn, skip)


def _trilinear(vol, pts):
    C, D, H, W = vol.shape
    dims = jnp.array([D - 1, H - 1, W - 1], dtype=pts.dtype)
    coords = jnp.clip(pts, 0.0, 1.0) * dims
    c0 = jnp.floor(coords).astype(jnp.int32)
    c0 = jnp.clip(c0, 0, jnp.array([D - 2, H - 2, W - 2], dtype=jnp.int32))
    frac = coords - c0.astype(pts.dtype)
    x0, y0, z0 = c0[:, 0], c0[:, 1], c0[:, 2]
    x1, y1, z1 = x0 + 1, y0 + 1, z0 + 1
    fx, fy, fz = frac[:, 0:1], frac[:, 1:2], frac[:, 2:3]

    def g(a, b, c):
        return vol[:, a, b, c].T

    return (g(x0, y0, z0) * (1 - fx) * (1 - fy) * (1 - fz)
            + g(x1, y0, z0) * fx * (1 - fy) * (1 - fz)
            + g(x0, y1, z0) * (1 - fx) * fy * (1 - fz)
            + g(x0, y0, z1) * (1 - fx) * (1 - fy) * fz
            + g(x1, y1, z0) * fx * fy * (1 - fz)
            + g(x1, y0, z1) * fx * (1 - fy) * fz
            + g(x0, y1, z1) * (1 - fx) * fy * fz
            + g(x1, y1, z1) * fx * fy * fz)


def kernel(verts, verts_features, skips, edge_index, params):
    b, v, d = verts.shape
    src = edge_index[0]
    dst = edge_index[1]
    vp = verts.reshape(V, d)
    fp = verts_features.reshape(V, -1)

    x0 = jnp.concatenate([vp, fp], axis=-1)

    t16 = jnp.concatenate(
        [x0, jnp.ones((V, 1), jnp.float32), jnp.zeros((V, 7), jnp.float32)],
        axis=-1)
    agg16 = _seg_mean_sum(t16, src, dst)
    deg = agg16[:, 8:9]
    invd = 1.0 / jnp.maximum(deg, 1.0)

    ws1, wn1 = params['g0_convs'][0]
    wn1p = jnp.concatenate([wn1, jnp.zeros((8, HID), jnp.float32)], axis=0)
    h = _combine(x0, agg16, invd, ws1, wn1p)
    for (ws, wn) in params['g0_convs'][1:-1]:
        h = _combine(h, _seg_mean_sum(h, src, dst), invd, ws, wn)
    ws3, wn3 = params['g0_convs'][-1]
    latent = _combine_res(h, _seg_mean_sum(h, src, dst), invd, ws3, wn3,
                          x0 @ params['g0_skip'])

    h_step = 1.0 / N_EULER
    vpt = vp
    for _ in range(N_EULER):
        skipped = _trilinear(skips[0], vpt)
        xd = jnp.concatenate([vpt, latent, skipped], axis=-1)
        ws, wn = params['d0_convs'][0]
        t = xd @ wn
        h = _combine(xd, _seg_mean_sum(t, src, dst), invd, ws,
                     jnp.eye(HID, dtype=jnp.float32))
        for (ws, wn) in params['d0_convs'][1:-1]:
            h = _combine(h, _seg_mean_sum(h, src, dst), invd, ws, wn)
        ws3, wn3 = params['d0_convs'][-1]
        h = _combine_res(h, _seg_mean_sum(h, src, dst), invd, ws3, wn3,
                         xd @ params['d0_skip'])
        for convs in (params['d1_convs'], params['d2_convs']):
            x_in = h
            for (ws, wn) in convs[:-1]:
                h = _combine(h, _seg_mean_sum(h, src, dst), invd, ws, wn)
            ws3, wn3 = convs[-1]
            h = _combine_res(h, _seg_mean_sum(h, src, dst), invd, ws3, wn3,
                             x_in)
        t3 = h @ params['f2v_n']
        t16v = jnp.concatenate([t3, jnp.zeros((V, 13), jnp.float32)], axis=-1)
        s = _seg_mean_sum(t16v, src, dst)
        dV = h @ params['f2v_s'] + (s * invd)[:, :3]
        vpt = vpt + h_step * dV

    return vpt.reshape(b, v, d)


# speedup vs baseline: 2.5781x; 2.3968x over previous
import functools

import jax
import jax.numpy as jnp
from jax import lax
from jax.experimental import pallas as pl
from jax.experimental.pallas import tpu as pltpu
from jax.experimental.pallas import tpu_sc as plsc

V = 10000
E = 320000
HID = 128
N_EULER = 2

NC = 2
NS = 16
NW = NC * NS

VP = 10240
EP = 327680
EPT = EP // NW
KE = 128
NCH = EPT // KE

RB = 640
GRIDV = VP // RB

def _mesh():
    return plsc.VectorSubcoreMesh(core_axis_name="c", subcore_axis_name="s",
                                  num_cores=NC, num_subcores=NS)


def _segsum_body(table, src, dst, out, src_v, dst_v, rows_v, acc, sem):
    C = rows_v.shape[1]
    cid = lax.axis_index("c")
    sid = lax.axis_index("s")
    wid = sid * NC + cid

    def zloop(i, _):
        for j in range(C // 16):
            rows_v[i, pl.ds(j * 16, 16)] = jnp.zeros((16,), jnp.float32)
        return _
    lax.fori_loop(0, KE, zloop, 0)
    rpt = VP // NS
    for z in range(rpt // KE):
        pltpu.sync_copy(rows_v, acc.at[pl.ds(sid * rpt + z * KE, KE)])
    plsc.subcore_barrier()

    def body(j, _):
        base = wid * EPT + j * KE
        pltpu.sync_copy(src.at[pl.ds(base, KE)], src_v)
        pltpu.sync_copy(dst.at[pl.ds(base, KE)], dst_v)
        pltpu.async_copy(table.at[src_v], rows_v, sem).wait()
        pltpu.sync_copy(rows_v, acc.at[dst_v], add=True)
        return _
    lax.fori_loop(0, NCH, body, 0)
    plsc.subcore_barrier()

    for z in range(rpt // KE):
        r0 = sid * rpt + z * KE
        pltpu.sync_copy(acc.at[pl.ds(r0, KE)], rows_v)
        pltpu.sync_copy(rows_v, out.at[pl.ds(cid * VP + r0, KE)])


@functools.lru_cache(maxsize=None)
def _make_segsum(C):
    return pl.kernel(
        _segsum_body,
        out_type=jax.ShapeDtypeStruct((2 * VP, C), jnp.float32),
        mesh=_mesh(),
        scratch_types=[
            pltpu.VMEM((KE,), jnp.int32),
            pltpu.VMEM((KE,), jnp.int32),
            pltpu.VMEM((KE, C), jnp.float32),
            pltpu.VMEM_SHARED((VP, C), jnp.float32),
            pltpu.SemaphoreType.DMA,
        ],
    )


def _segsum128(*a):
    return _make_segsum(128)(*a)

NG = 8 * VP
NGT = NG // NW
NGCH = NGT // KE


def _gather32_body(table, idx, out, idx_v, rows_v, sem):
    cid = lax.axis_index("c")
    sid = lax.axis_index("s")
    wid = sid * NC + cid

    def body(j, _):
        base = wid * NGT + j * KE
        pltpu.sync_copy(idx.at[pl.ds(base, KE)], idx_v)
        pltpu.async_copy(table.at[idx_v], rows_v, sem).wait()
        pltpu.sync_copy(rows_v, out.at[pl.ds(base, KE)])
        return _
    lax.fori_loop(0, NGCH, body, 0)


@functools.lru_cache(maxsize=None)
def _make_gather32():
    return pl.kernel(
        _gather32_body,
        out_type=jax.ShapeDtypeStruct((NG, 128), jnp.float32),
        mesh=_mesh(),
        scratch_types=[
            pltpu.VMEM((KE,), jnp.int32),
            pltpu.VMEM((KE, 128), jnp.float32),
            pltpu.SemaphoreType.DMA,
        ],
    )


def _gather32(*a):
    return _make_gather32()(*a)


def _rspec(c):
    return pl.BlockSpec((RB, c), lambda i: (i, 0))


def _rspec_hi(c):
    return pl.BlockSpec((RB, c), lambda i: (i + GRIDV, 0))


def _wspec(r, c):
    return pl.BlockSpec((r, c), lambda i: (0, 0))


def _vcall(body, out_c, specs, n_out=1):
    out_shape = jax.ShapeDtypeStruct((VP, out_c), jnp.float32)
    if n_out > 1:
        out_shape = [out_shape] * n_out
        out_specs = [_rspec(out_c)] * n_out
    else:
        out_specs = _rspec(out_c)
    return pl.pallas_call(body, grid=(GRIDV,), in_specs=specs,
                          out_specs=out_specs, out_shape=out_shape)


def _invd_body(a0, a1, o):
    deg = a0[:, 8:9] + a1[:, 8:9]
    o[...] = (1.0 / jnp.maximum(deg, 1.0)) * jnp.ones((1, 16), jnp.float32)


def _comb16_body(x, a0, a1, ivd, ws, wn, o):
    agg = (a0[:, 0:16] + a1[:, 0:16]) * ivd[:, 0:1]
    o[...] = jnp.maximum(
        jnp.dot(x[...], ws[...], preferred_element_type=jnp.float32)
        + jnp.dot(agg, wn[...], preferred_element_type=jnp.float32), 0.0)


def _comb_body(x, a0, a1, ivd, ws, wn, o):
    agg = (a0[...] + a1[...]) * ivd[:, 0:1]
    o[...] = jnp.maximum(
        jnp.dot(x[...], ws[...], preferred_element_type=jnp.float32)
        + jnp.dot(agg, wn[...], preferred_element_type=jnp.float32), 0.0)


def _comb_res_body(x, a0, a1, ivd, ws, wn, xs, wk, o):
    agg = (a0[...] + a1[...]) * ivd[:, 0:1]
    h = jnp.maximum(
        jnp.dot(x[...], ws[...], preferred_element_type=jnp.float32)
        + jnp.dot(agg, wn[...], preferred_element_type=jnp.float32), 0.0)
    o[...] = jnp.maximum(
        h + jnp.dot(xs[...], wk[...], preferred_element_type=jnp.float32), 0.0)


def _comb_res_id_body(x, a0, a1, ivd, ws, wn, skip, o):
    agg = (a0[...] + a1[...]) * ivd[:, 0:1]
    h = jnp.maximum(
        jnp.dot(x[...], ws[...], preferred_element_type=jnp.float32)
        + jnp.dot(agg, wn[...], preferred_element_type=jnp.float32), 0.0)
    o[...] = jnp.maximum(h + skip[...], 0.0)


def _mm3_body(a, b, c, wa, wb, wc, o):
    o[...] = (jnp.dot(a[...], wa[...], preferred_element_type=jnp.float32)
              + jnp.dot(b[...], wb[...], preferred_element_type=jnp.float32)
              + jnp.dot(c[...], wc[...], preferred_element_type=jnp.float32))


def _comb3_body(a, b, c, wa, wb, wc, s0, s1, ivd, o):
    z = (jnp.dot(a[...], wa[...], preferred_element_type=jnp.float32)
         + jnp.dot(b[...], wb[...], preferred_element_type=jnp.float32)
         + jnp.dot(c[...], wc[...], preferred_element_type=jnp.float32)
         + (s0[...] + s1[...]) * ivd[:, 0:1])
    o[...] = jnp.maximum(z, 0.0)


def _comb_res3_body(x, a0, a1, ivd, ws, wn, xa, xb, xc, wka, wkb, wkc, o):
    agg = (a0[...] + a1[...]) * ivd[:, 0:1]
    h = jnp.maximum(
        jnp.dot(x[...], ws[...], preferred_element_type=jnp.float32)
        + jnp.dot(agg, wn[...], preferred_element_type=jnp.float32), 0.0)
    skip = (jnp.dot(xa[...], wka[...], preferred_element_type=jnp.float32)
            + jnp.dot(xb[...], wkb[...], preferred_element_type=jnp.float32)
            + jnp.dot(xc[...], wkc[...], preferred_element_type=jnp.float32))
    o[...] = jnp.maximum(h + skip, 0.0)


def _tri_prep_body(p, idx_o, w_o):
    pts = jnp.clip(p[...][:, 0:3], 0.0, 1.0) * 31.0
    c0 = jnp.clip(jnp.floor(pts).astype(jnp.int32), 0, 30)
    frac = pts - c0.astype(jnp.float32)
    x0 = c0[:, 0:1]
    y0 = c0[:, 1:2]
    z0 = c0[:, 2:3]
    fx = frac[:, 0:1]
    fy = frac[:, 1:2]
    fz = frac[:, 2:3]
    idx_cols = []
    w_cols = []
    for dx in (0, 1):
        for dy in (0, 1):
            for dz in (0, 1):
                ix = ((x0 + dx) * 32 + (y0 + dy)) * 32 + (z0 + dz)
                wx = fx if dx else (1.0 - fx)
                wy = fy if dy else (1.0 - fy)
                wz = fz if dz else (1.0 - fz)
                idx_cols.append(ix)
                w_cols.append(wx * wy * wz)
    idx_o[...] = jnp.concatenate(idx_cols, axis=1)
    w_o[...] = jnp.concatenate(w_cols, axis=1)


def _skip_body(c_ref, w_ref, o):
    acc = c_ref[:, 0, 0:32] * w_ref[:, 0:1]
    for k in range(1, 8):
        acc = acc + c_ref[:, k, 0:32] * w_ref[:, k:k + 1]
    o[...] = acc


def _f2v_pre_body(h, wn, o):
    o[...] = jnp.dot(h[...], wn[...], preferred_element_type=jnp.float32)


def _euler_body(vpt, h, ws, s0, s1, ivd, o):
    hm = jnp.dot(h[...], ws[...], preferred_element_type=jnp.float32)
    dv = hm + (s0[:, 0:16] + s1[:, 0:16]) * ivd[:, 0:1]
    o[...] = vpt[...] + (1.0 / N_EULER) * dv[:, 0:8]


def _pad2(w, r, c):
    return jnp.pad(w, ((0, r - w.shape[0]), (0, c - w.shape[1])))


def kernel(verts, verts_features, skips, edge_index, params):
    b, v, d = verts.shape
    src = edge_index[0]
    dst = edge_index[1]

    srcp = jnp.concatenate([src, jnp.zeros((EP - E,), jnp.int32)])
    dstp = jnp.concatenate([dst, jnp.full((EP - E,), VP - 1, jnp.int32)])
    vp = verts.reshape(V, d)
    fp = verts_features.reshape(V, -1)
    vp8 = jnp.pad(vp, ((0, VP - V), (0, 8 - d)))
    x0 = jnp.concatenate([vp, fp], axis=-1)
    x0p = jnp.pad(x0, ((0, VP - V), (0, 8)))
    t128 = jnp.pad(jnp.concatenate([x0, jnp.ones((V, 1), jnp.float32)],
                                   axis=-1), ((0, VP - V), (0, 119)))
    vol = jnp.pad(
        jnp.transpose(skips[0], (1, 2, 3, 0)).reshape(32 * 32 * 32, 32),
        ((0, 0), (0, 96)))

    (ws1, wn1), (ws2, wn2), (ws3, wn3) = params['g0_convs']
    ws1p = _pad2(ws1, 16, HID)
    wn1p = _pad2(wn1, 16, HID)
    wg0k = _pad2(params['g0_skip'], 16, HID)
    dws1, dwn1 = params['d0_convs'][0]
    dws1v, dws1l, dws1s = dws1[0:3], dws1[3:131], dws1[131:163]
    dwn1v, dwn1l, dwn1s = dwn1[0:3], dwn1[3:131], dwn1[131:163]
    dws1v = _pad2(dws1v, 8, HID)
    dwn1v = _pad2(dwn1v, 8, HID)
    d0k = params['d0_skip']
    d0kv, d0kl, d0ks = _pad2(d0k[0:3], 8, HID), d0k[3:131], d0k[131:163]
    f2vn = _pad2(params['f2v_n'], HID, HID)
    f2vs = _pad2(params['f2v_s'], HID, 16)

    agg = _segsum128(t128, srcp, dstp)
    invd = _vcall(_invd_body, 16, [_rspec(128), _rspec_hi(128)])(agg, agg)

    comb16 = _vcall(_comb16_body, 128,
                    [_rspec(16), _rspec(128), _rspec_hi(128), _rspec(16),
                     _wspec(16, HID), _wspec(16, HID)])
    comb128 = _vcall(_comb_body, 128,
                     [_rspec(128), _rspec(128), _rspec_hi(128), _rspec(128),
                      _wspec(HID, HID), _wspec(HID, HID)])
    comb_res16 = _vcall(_comb_res_body, 128,
                        [_rspec(128), _rspec(128), _rspec_hi(128),
                         _rspec(128), _wspec(HID, HID), _wspec(HID, HID),
                         _rspec(16), _wspec(16, HID)])
    comb_res_id = _vcall(_comb_res_id_body, 128,
                         [_rspec(128), _rspec(128), _rspec_hi(128),
                          _rspec(128), _wspec(HID, HID), _wspec(HID, HID),
                          _rspec(128)])
    mm3 = _vcall(_mm3_body, 128,
                 [_rspec(8), _rspec(128), _rspec(32), _wspec(8, HID),
                  _wspec(HID, HID), _wspec(32, HID)])
    comb3 = _vcall(_comb3_body, 128,
                   [_rspec(8), _rspec(128), _rspec(32), _wspec(8, HID),
                    _wspec(HID, HID), _wspec(32, HID), _rspec(128),
                    _rspec_hi(128), _rspec(128)])
    comb_res3 = _vcall(_comb_res3_body, 128,
                       [_rspec(128), _rspec(128), _rspec_hi(128), _rspec(128),
                        _wspec(HID, HID), _wspec(HID, HID), _rspec(8),
                        _rspec(128), _rspec(32), _wspec(8, HID),
                        _wspec(HID, HID), _wspec(32, HID)])
    tri_prep = pl.pallas_call(
        _tri_prep_body, grid=(GRIDV,), in_specs=[_rspec(8)],
        out_specs=[_rspec(8), _rspec(8)],
        out_shape=[jax.ShapeDtypeStruct((VP, 8), jnp.int32),
                   jax.ShapeDtypeStruct((VP, 8), jnp.float32)])
    skipc = pl.pallas_call(
        _skip_body, grid=(GRIDV,),
        in_specs=[pl.BlockSpec((RB, 8, 128), lambda i: (i, 0, 0)), _rspec(8)],
        out_specs=_rspec(32),
        out_shape=jax.ShapeDtypeStruct((VP, 32), jnp.float32))
    f2v_pre = _vcall(_f2v_pre_body, 128,
                     [_rspec(128), _wspec(HID, HID)])
    euler = _vcall(_euler_body, 8,
                   [_rspec(8), _rspec(128), _wspec(HID, 16), _rspec(128),
                    _rspec_hi(128), _rspec(16)])

    h = comb16(x0p, agg, agg, invd, ws1p, wn1p)
    s = _segsum128(h, srcp, dstp)
    h = comb128(h, s, s, invd, ws2, wn2)
    s = _segsum128(h, srcp, dstp)
    latent = comb_res16(h, s, s, invd, ws3, wn3, x0p, wg0k)

    vpt8 = vp8
    for _ in range(N_EULER):
        idx, w = tri_prep(vpt8)
        corners = _gather32(vol, idx.reshape(NG))
        skipped = skipc(corners.reshape(VP, 8, 128), w)
        t = mm3(vpt8, latent, skipped, dwn1v, dwn1l, dwn1s)
        s = _segsum128(t, srcp, dstp)
        h = comb3(vpt8, latent, skipped, dws1v, dws1l, dws1s, s, s, invd)
        ws_, wn_ = params['d0_convs'][1]
        s = _segsum128(h, srcp, dstp)
        h = comb128(h, s, s, invd, ws_, wn_)
        ws_, wn_ = params['d0_convs'][2]
        s = _segsum128(h, srcp, dstp)
        h = comb_res3(h, s, s, invd, ws_, wn_, vpt8, latent, skipped,
                      d0kv, d0kl, d0ks)
        for convs in (params['d1_convs'], params['d2_convs']):
            xin = h
            for (ws_, wn_) in convs[:-1]:
                s = _segsum128(h, srcp, dstp)
                h = comb128(h, s, s, invd, ws_, wn_)
            ws_, wn_ = convs[-1]
            s = _segsum128(h, srcp, dstp)
            h = comb_res_id(h, s, s, invd, ws_, wn_, xin)
        tf = f2v_pre(h, f2vn)
        s = _segsum128(tf, srcp, dstp)
        vpt8 = euler(vpt8, h, f2vs, s, s, invd)

    return vpt8[:V, :3].reshape(b, v, d)


# speedup vs baseline: 3.3767x; 1.3098x over previous
import functools

import jax
import jax.numpy as jnp
from jax import lax
from jax.experimental import pallas as pl
from jax.experimental.pallas import tpu as pltpu
from jax.experimental.pallas import tpu_sc as plsc

V = 10000
E = 320000
HID = 128
N_EULER = 2

NC = 2
NS = 16
NW = NC * NS

VP = 10240
EP = 327680
EPT = EP // NW
KE = 64
NCH = EPT // KE

RB = 640
GRIDV = VP // RB

def _mesh():
    return plsc.VectorSubcoreMesh(core_axis_name="c", subcore_axis_name="s",
                                  num_cores=NC, num_subcores=NS)


def _segsum_body(table, src3, dst3, out, src_all, dst_v0, dst_v1, rows0,
                 rows1, acc, sem0, sem1, semd0, semd1):
    C = rows0.shape[1]
    cid = lax.axis_index("c")
    sid = lax.axis_index("s")
    wid = sid * NC + cid

    pltpu.sync_copy(src3.at[wid], src_all)

    def zloop(i, _):
        for j in range(C // 16):
            rows0[i, pl.ds(j * 16, 16)] = jnp.zeros((16,), jnp.float32)
        return _
    lax.fori_loop(0, KE, zloop, 0)
    rpt = VP // NS
    for z in range(rpt // KE):
        pltpu.sync_copy(rows0, acc.at[pl.ds(sid * rpt + z * KE, KE)])
    plsc.subcore_barrier()

    pltpu.async_copy(table.at[src_all.at[0]], rows0, sem0)
    pltpu.async_copy(dst3.at[wid, 0], dst_v0, semd0)

    def body(i, _):
        j0 = 2 * i
        pltpu.async_copy(table.at[src_all.at[j0 + 1]], rows1, sem1)
        pltpu.async_copy(dst3.at[wid, j0 + 1], dst_v1, semd1)
        pltpu.make_async_copy(table.at[src_all.at[j0]], rows0, sem0).wait()
        pltpu.make_async_copy(dst3.at[wid, j0], dst_v0, semd0).wait()
        pltpu.sync_copy(rows0, acc.at[dst_v0], add=True)

        @pl.when(i < NCH // 2 - 1)
        def _fire_next():
            pltpu.async_copy(table.at[src_all.at[j0 + 2]], rows0, sem0)
            pltpu.async_copy(dst3.at[wid, j0 + 2], dst_v0, semd0)

        pltpu.make_async_copy(table.at[src_all.at[j0 + 1]], rows1,
                              sem1).wait()
        pltpu.make_async_copy(dst3.at[wid, j0 + 1], dst_v1, semd1).wait()
        pltpu.sync_copy(rows1, acc.at[dst_v1], add=True)
        return _
    lax.fori_loop(0, NCH // 2, body, 0)
    plsc.subcore_barrier()

    for z in range(rpt // KE):
        r0 = sid * rpt + z * KE
        pltpu.sync_copy(acc.at[pl.ds(r0, KE)], rows0)
        pltpu.sync_copy(rows0, out.at[pl.ds(cid * VP + r0, KE)])


@functools.lru_cache(maxsize=None)
def _make_segsum(C):
    return pl.kernel(
        _segsum_body,
        out_type=jax.ShapeDtypeStruct((2 * VP, C), jnp.float32),
        mesh=_mesh(),
        scratch_types=[
            pltpu.VMEM((NCH, KE), jnp.int32),
            pltpu.VMEM((KE,), jnp.int32),
            pltpu.VMEM((KE,), jnp.int32),
            pltpu.VMEM((KE, C), jnp.float32),
            pltpu.VMEM((KE, C), jnp.float32),
            pltpu.VMEM_SHARED((VP, C), jnp.float32),
            pltpu.SemaphoreType.DMA,
            pltpu.SemaphoreType.DMA,
            pltpu.SemaphoreType.DMA,
            pltpu.SemaphoreType.DMA,
        ],
    )


def _segsum128(*a):
    return _make_segsum(128)(*a)

NG = 8 * VP
NGT = NG // NW
NGCH = NGT // KE


def _gather32_body(table, idx3, out, idx_all, rows0, rows1, sem0, sem1):
    cid = lax.axis_index("c")
    sid = lax.axis_index("s")
    wid = sid * NC + cid

    pltpu.sync_copy(idx3.at[wid], idx_all)
    pltpu.async_copy(table.at[idx_all.at[0]], rows0, sem0)

    def body(i, _):
        j0 = 2 * i
        base = wid * NGT + j0 * KE
        pltpu.async_copy(table.at[idx_all.at[j0 + 1]], rows1, sem1)
        pltpu.make_async_copy(table.at[idx_all.at[j0]], rows0, sem0).wait()
        pltpu.sync_copy(rows0, out.at[pl.ds(base, KE)])

        @pl.when(i < NGCH // 2 - 1)
        def _fire_next():
            pltpu.async_copy(table.at[idx_all.at[j0 + 2]], rows0, sem0)

        pltpu.make_async_copy(table.at[idx_all.at[j0 + 1]], rows1,
                              sem1).wait()
        pltpu.sync_copy(rows1, out.at[pl.ds(base + KE, KE)])
        return _
    lax.fori_loop(0, NGCH // 2, body, 0)


@functools.lru_cache(maxsize=None)
def _make_gather32():
    return pl.kernel(
        _gather32_body,
        out_type=jax.ShapeDtypeStruct((NG, 128), jnp.float32),
        mesh=_mesh(),
        scratch_types=[
            pltpu.VMEM((NGCH, KE), jnp.int32),
            pltpu.VMEM((KE, 128), jnp.float32),
            pltpu.VMEM((KE, 128), jnp.float32),
            pltpu.SemaphoreType.DMA,
            pltpu.SemaphoreType.DMA,
        ],
    )


def _gather32(*a):
    return _make_gather32()(*a)


def _rspec(c):
    return pl.BlockSpec((RB, c), lambda i: (i, 0))


def _rspec_hi(c):
    return pl.BlockSpec((RB, c), lambda i: (i + GRIDV, 0))


def _wspec(r, c):
    return pl.BlockSpec((r, c), lambda i: (0, 0))


def _vcall(body, out_c, specs, n_out=1):
    out_shape = jax.ShapeDtypeStruct((VP, out_c), jnp.float32)
    if n_out > 1:
        out_shape = [out_shape] * n_out
        out_specs = [_rspec(out_c)] * n_out
    else:
        out_specs = _rspec(out_c)
    return pl.pallas_call(body, grid=(GRIDV,), in_specs=specs,
                          out_specs=out_specs, out_shape=out_shape)


def _invd_body(a0, a1, o):
    deg = a0[:, 8:9] + a1[:, 8:9]
    o[...] = (1.0 / jnp.maximum(deg, 1.0)) * jnp.ones((1, 16), jnp.float32)


def _comb16_body(x, a0, a1, ivd, ws, wn, o):
    agg = (a0[:, 0:16] + a1[:, 0:16]) * ivd[:, 0:1]
    o[...] = jnp.maximum(
        jnp.dot(x[...], ws[...], preferred_element_type=jnp.float32)
        + jnp.dot(agg, wn[...], preferred_element_type=jnp.float32), 0.0)


def _comb_body(x, a0, a1, ivd, ws, wn, o):
    agg = (a0[...] + a1[...]) * ivd[:, 0:1]
    o[...] = jnp.maximum(
        jnp.dot(x[...], ws[...], preferred_element_type=jnp.float32)
        + jnp.dot(agg, wn[...], preferred_element_type=jnp.float32), 0.0)


def _comb_res_body(x, a0, a1, ivd, ws, wn, xs, wk, o):
    agg = (a0[...] + a1[...]) * ivd[:, 0:1]
    h = jnp.maximum(
        jnp.dot(x[...], ws[...], preferred_element_type=jnp.float32)
        + jnp.dot(agg, wn[...], preferred_element_type=jnp.float32), 0.0)
    o[...] = jnp.maximum(
        h + jnp.dot(xs[...], wk[...], preferred_element_type=jnp.float32), 0.0)


def _comb_res_id_body(x, a0, a1, ivd, ws, wn, skip, o):
    agg = (a0[...] + a1[...]) * ivd[:, 0:1]
    h = jnp.maximum(
        jnp.dot(x[...], ws[...], preferred_element_type=jnp.float32)
        + jnp.dot(agg, wn[...], preferred_element_type=jnp.float32), 0.0)
    o[...] = jnp.maximum(h + skip[...], 0.0)


def _mm3_body(a, b, c, wa, wb, wc, o):
    o[...] = (jnp.dot(a[...], wa[...], preferred_element_type=jnp.float32)
              + jnp.dot(b[...], wb[...], preferred_element_type=jnp.float32)
              + jnp.dot(c[...], wc[...], preferred_element_type=jnp.float32))


def _comb3_body(a, b, c, wa, wb, wc, s0, s1, ivd, o):
    z = (jnp.dot(a[...], wa[...], preferred_element_type=jnp.float32)
         + jnp.dot(b[...], wb[...], preferred_element_type=jnp.float32)
         + jnp.dot(c[...], wc[...], preferred_element_type=jnp.float32)
         + (s0[...] + s1[...]) * ivd[:, 0:1])
    o[...] = jnp.maximum(z, 0.0)


def _comb_res3_body(x, a0, a1, ivd, ws, wn, xa, xb, xc, wka, wkb, wkc, o):
    agg = (a0[...] + a1[...]) * ivd[:, 0:1]
    h = jnp.maximum(
        jnp.dot(x[...], ws[...], preferred_element_type=jnp.float32)
        + jnp.dot(agg, wn[...], preferred_element_type=jnp.float32), 0.0)
    skip = (jnp.dot(xa[...], wka[...], preferred_element_type=jnp.float32)
            + jnp.dot(xb[...], wkb[...], preferred_element_type=jnp.float32)
            + jnp.dot(xc[...], wkc[...], preferred_element_type=jnp.float32))
    o[...] = jnp.maximum(h + skip, 0.0)


def _tri_prep_body(p, idx_o, w_o):
    pts = jnp.clip(p[...][:, 0:3], 0.0, 1.0) * 31.0
    c0 = jnp.clip(jnp.floor(pts).astype(jnp.int32), 0, 30)
    frac = pts - c0.astype(jnp.float32)
    x0 = c0[:, 0:1]
    y0 = c0[:, 1:2]
    z0 = c0[:, 2:3]
    fx = frac[:, 0:1]
    fy = frac[:, 1:2]
    fz = frac[:, 2:3]
    idx_cols = []
    w_cols = []
    for dx in (0, 1):
        for dy in (0, 1):
            for dz in (0, 1):
                ix = ((x0 + dx) * 32 + (y0 + dy)) * 32 + (z0 + dz)
                wx = fx if dx else (1.0 - fx)
                wy = fy if dy else (1.0 - fy)
                wz = fz if dz else (1.0 - fz)
                idx_cols.append(ix)
                w_cols.append(wx * wy * wz)
    idx_o[...] = jnp.concatenate(idx_cols, axis=1)
    w_o[...] = jnp.concatenate(w_cols, axis=1)


def _skip_body(c_ref, w_ref, o):
    acc = c_ref[:, 0, 0:32] * w_ref[:, 0:1]
    for k in range(1, 8):
        acc = acc + c_ref[:, k, 0:32] * w_ref[:, k:k + 1]
    o[...] = acc


def _f2v_pre_body(h, wn, o):
    o[...] = jnp.dot(h[...], wn[...], preferred_element_type=jnp.float32)


def _euler_body(vpt, h, ws, s0, s1, ivd, o):
    hm = jnp.dot(h[...], ws[...], preferred_element_type=jnp.float32)
    dv = hm + (s0[:, 0:16] + s1[:, 0:16]) * ivd[:, 0:1]
    o[...] = vpt[...] + (1.0 / N_EULER) * dv[:, 0:8]


def _pad2(w, r, c):
    return jnp.pad(w, ((0, r - w.shape[0]), (0, c - w.shape[1])))


def kernel(verts, verts_features, skips, edge_index, params):
    b, v, d = verts.shape
    src = edge_index[0]
    dst = edge_index[1]

    srcp = jnp.concatenate(
        [src, jnp.zeros((EP - E,), jnp.int32)]).reshape(NW, NCH, KE)
    dstp = jnp.concatenate(
        [dst, jnp.full((EP - E,), VP - 1, jnp.int32)]).reshape(NW, NCH, KE)
    vp = verts.reshape(V, d)
    fp = verts_features.reshape(V, -1)
    vp8 = jnp.pad(vp, ((0, VP - V), (0, 8 - d)))
    x0 = jnp.concatenate([vp, fp], axis=-1)
    x0p = jnp.pad(x0, ((0, VP - V), (0, 8)))
    t128 = jnp.pad(jnp.concatenate([x0, jnp.ones((V, 1), jnp.float32)],
                                   axis=-1), ((0, VP - V), (0, 119)))
    vol = jnp.pad(
        jnp.transpose(skips[0], (1, 2, 3, 0)).reshape(32 * 32 * 32, 32),
        ((0, 0), (0, 96)))

    (ws1, wn1), (ws2, wn2), (ws3, wn3) = params['g0_convs']
    ws1p = _pad2(ws1, 16, HID)
    wn1p = _pad2(wn1, 16, HID)
    wg0k = _pad2(params['g0_skip'], 16, HID)
    dws1, dwn1 = params['d0_convs'][0]
    dws1v, dws1l, dws1s = dws1[0:3], dws1[3:131], dws1[131:163]
    dwn1v, dwn1l, dwn1s = dwn1[0:3], dwn1[3:131], dwn1[131:163]
    dws1v = _pad2(dws1v, 8, HID)
    dwn1v = _pad2(dwn1v, 8, HID)
    d0k = params['d0_skip']
    d0kv, d0kl, d0ks = _pad2(d0k[0:3], 8, HID), d0k[3:131], d0k[131:163]
    f2vn = _pad2(params['f2v_n'], HID, HID)
    f2vs = _pad2(params['f2v_s'], HID, 16)

    agg = _segsum128(t128, srcp, dstp)
    invd = _vcall(_invd_body, 16, [_rspec(128), _rspec_hi(128)])(agg, agg)

    comb16 = _vcall(_comb16_body, 128,
                    [_rspec(16), _rspec(128), _rspec_hi(128), _rspec(16),
                     _wspec(16, HID), _wspec(16, HID)])
    comb128 = _vcall(_comb_body, 128,
                     [_rspec(128), _rspec(128), _rspec_hi(128), _rspec(128),
                      _wspec(HID, HID), _wspec(HID, HID)])
    comb_res16 = _vcall(_comb_res_body, 128,
                        [_rspec(128), _rspec(128), _rspec_hi(128),
                         _rspec(128), _wspec(HID, HID), _wspec(HID, HID),
                         _rspec(16), _wspec(16, HID)])
    comb_res_id = _vcall(_comb_res_id_body, 128,
                         [_rspec(128), _rspec(128), _rspec_hi(128),
                          _rspec(128), _wspec(HID, HID), _wspec(HID, HID),
                          _rspec(128)])
    mm3 = _vcall(_mm3_body, 128,
                 [_rspec(8), _rspec(128), _rspec(32), _wspec(8, HID),
                  _wspec(HID, HID), _wspec(32, HID)])
    comb3 = _vcall(_comb3_body, 128,
                   [_rspec(8), _rspec(128), _rspec(32), _wspec(8, HID),
                    _wspec(HID, HID), _wspec(32, HID), _rspec(128),
                    _rspec_hi(128), _rspec(128)])
    comb_res3 = _vcall(_comb_res3_body, 128,
                       [_rspec(128), _rspec(128), _rspec_hi(128), _rspec(128),
                        _wspec(HID, HID), _wspec(HID, HID), _rspec(8),
                        _rspec(128), _rspec(32), _wspec(8, HID),
                        _wspec(HID, HID), _wspec(32, HID)])
    tri_prep = pl.pallas_call(
        _tri_prep_body, grid=(GRIDV,), in_specs=[_rspec(8)],
        out_specs=[_rspec(8), _rspec(8)],
        out_shape=[jax.ShapeDtypeStruct((VP, 8), jnp.int32),
                   jax.ShapeDtypeStruct((VP, 8), jnp.float32)])
    skipc = pl.pallas_call(
        _skip_body, grid=(GRIDV,),
        in_specs=[pl.BlockSpec((RB, 8, 128), lambda i: (i, 0, 0)), _rspec(8)],
        out_specs=_rspec(32),
        out_shape=jax.ShapeDtypeStruct((VP, 32), jnp.float32))
    f2v_pre = _vcall(_f2v_pre_body, 128,
                     [_rspec(128), _wspec(HID, HID)])
    euler = _vcall(_euler_body, 8,
                   [_rspec(8), _rspec(128), _wspec(HID, 16), _rspec(128),
                    _rspec_hi(128), _rspec(16)])

    h = comb16(x0p, agg, agg, invd, ws1p, wn1p)
    s = _segsum128(h, srcp, dstp)
    h = comb128(h, s, s, invd, ws2, wn2)
    s = _segsum128(h, srcp, dstp)
    latent = comb_res16(h, s, s, invd, ws3, wn3, x0p, wg0k)

    vpt8 = vp8
    for _ in range(N_EULER):
        idx, w = tri_prep(vpt8)
        corners = _gather32(vol, idx.reshape(NW, NGCH, KE))
        skipped = skipc(corners.reshape(VP, 8, 128), w)
        t = mm3(vpt8, latent, skipped, dwn1v, dwn1l, dwn1s)
        s = _segsum128(t, srcp, dstp)
        h = comb3(vpt8, latent, skipped, dws1v, dws1l, dws1s, s, s, invd)
        ws_, wn_ = params['d0_convs'][1]
        s = _segsum128(h, srcp, dstp)
        h = comb128(h, s, s, invd, ws_, wn_)
        ws_, wn_ = params['d0_convs'][2]
        s = _segsum128(h, srcp, dstp)
        h = comb_res3(h, s, s, invd, ws_, wn_, vpt8, latent, skipped,
                      d0kv, d0kl, d0ks)
        for convs in (params['d1_convs'], params['d2_convs']):
            xin = h
            for (ws_, wn_) in convs[:-1]:
                s = _segsum128(h, srcp, dstp)
                h = comb128(h, s, s, invd, ws_, wn_)
            ws_, wn_ = convs[-1]
            s = _segsum128(h, srcp, dstp)
            h = comb_res_id(h, s, s, invd, ws_, wn_, xin)
        tf = f2v_pre(h, f2vn)
        s = _segsum128(tf, srcp, dstp)
        vpt8 = euler(vpt8, h, f2vs, s, s, invd)

    return vpt8[:V, :3].reshape(b, v, d)
